# baseline (speedup 1.0000x reference)
import functools
import math

import jax
import jax.numpy as jnp
import numpy as np
from jax import lax
from jax.experimental import pallas as pl
from jax.experimental.pallas import tpu as pltpu
from jax.experimental.pallas import tpu_sc as plsc

NN = 10000
EE = 320000
NUM_BASIS = 10
MAX_RADIUS = 5.0
INV_SQRT_NEIGH = 1.0 / math.sqrt(32.0)
BASIS_CONST = 1.14136 * float(np.exp(2.0)) * float(np.sqrt(NUM_BASIS))
INV_STEP = (NUM_BASIS + 1) / MAX_RADIUS
PAD_D2 = 1.0e6

NC, NS = 2, 16
NW = NC * NS
EP = 327680
EPT = EP // NW
IR = EPT // 128
CH = 128
IGRP = 4
NGRP = IR // IGRP
NSTAGE = 10
RPT = NN // NSTAGE

_SC_MESH = dict(core_axis_name="c", subcore_axis_name="s")
_SC_PARAMS = pltpu.CompilerParams(needs_layout_passes=False)
_TC_PARAMS = pltpu.CompilerParams(skip_device_barrier=True)


def _d2_body(pxi, pyi, pzi, pxf, pyf, pzf, pxt, pyt, pzt, src_hbm, dst_hbm,
             d2i_hbm, d2f_hbm, d2t_hbm,
             px_v, py_v, pz_v, src_v, dst_v, d2_v):
    c = lax.axis_index("c")
    s = lax.axis_index("s")
    wid = c * NS + s
    base = wid * EPT
    pltpu.sync_copy(src_hbm.at[pl.ds(wid * IR, IR), :], src_v)
    pltpu.sync_copy(dst_hbm.at[pl.ds(wid * IR, IR), :], dst_v)
    for (px, py, pz, d2_hbm) in ((pxi, pyi, pzi, d2i_hbm),
                                 (pxf, pyf, pzf, d2f_hbm),
                                 (pxt, pyt, pzt, d2t_hbm)):
        pltpu.sync_copy(px, px_v)
        pltpu.sync_copy(py, py_v)
        pltpu.sync_copy(pz, pz_v)

        @pl.loop(0, IR)
        def _row(r):
            @pl.loop(0, 8)
            def _grp(q):
                si = src_v[r, pl.ds(q * 16, 16)]
                di = dst_v[r, pl.ds(q * 16, 16)]
                dx = plsc.load_gather(px_v, [si]) - plsc.load_gather(px_v, [di])
                dy = plsc.load_gather(py_v, [si]) - plsc.load_gather(py_v, [di])
                dz = plsc.load_gather(pz_v, [si]) - plsc.load_gather(pz_v, [di])
                d2 = dx * dx + dy * dy + dz * dz
                off = r * 128 + q * 16
                gidx = lax.iota(jnp.int32, 16) + (base + off)
                d2_v[pl.ds(off, 16)] = jnp.where(gidx >= EE, PAD_D2, d2)

        pltpu.sync_copy(d2_v, d2_hbm.at[pl.ds(base, EPT)])


_d2_call = pl.kernel(
    _d2_body,
    out_type=[jax.ShapeDtypeStruct((EP,), jnp.float32)] * 3,
    mesh=plsc.VectorSubcoreMesh(**_SC_MESH),
    compiler_params=_SC_PARAMS,
    scratch_types=[
        pltpu.VMEM((NN,), jnp.float32),
        pltpu.VMEM((NN,), jnp.float32),
        pltpu.VMEM((NN,), jnp.float32),
        pltpu.VMEM((IR, 128), jnp.int32),
        pltpu.VMEM((IR, 128), jnp.int32),
        pltpu.VMEM((EPT,), jnp.float32),
    ],
)


def _gms_body(y_hbm, w_hbm, src_hbm, dst_hbm, zeros_hbm, agg_hbm,
              agg_sh, src_v, dst_v, w_a, w_b, rows_a, rows_b, gsem, ssem, wsem):
    c = lax.axis_index("c")
    s = lax.axis_index("s")
    wid = c * NS + s
    base = wid * EPT
    rbase = s * RPT
    @pl.when(s < NSTAGE)
    def _zero():
        pltpu.sync_copy(zeros_hbm.at[pl.ds(rbase, RPT), :],
                        agg_sh.at[pl.ds(rbase, RPT), :])
    plsc.subcore_barrier()

    rows = (rows_a, rows_b)
    wv = (w_a, w_b)
    nchunk = IR

    def stage_idx(g):
        pltpu.sync_copy(src_hbm.at[pl.ds(wid * IR + g * IGRP, IGRP), :], src_v)
        pltpu.sync_copy(dst_hbm.at[pl.ds(wid * IR + g * IGRP, IGRP), :],
                        dst_v.at[pl.ds((g % 2) * IGRP, IGRP), :])

    def gather(t):
        return pltpu.async_copy(y_hbm.at[src_v.at[t % IGRP]], rows[t % 2], gsem)

    def dst_row(t):
        return dst_v.at[((t // IGRP) % 2) * IGRP + (t % IGRP)]

    def wstream(t):
        off = pl.multiple_of(base // 2 + t * (CH // 2), 8)
        return pltpu.async_copy(
            w_hbm.at[pl.ds(off, CH // 2), :], wv[t % 2], wsem)

    stage_idx(0)
    desc_w = wstream(0)
    desc_g = gather(0)
    desc_s = {}
    for t in range(nchunk):
        desc_w.wait()
        desc_g.wait()
        nt = t + 1
        if nt < nchunk:
            if nt % IGRP == 0:
                stage_idx(nt // IGRP)
            desc_w = wstream(nt)
            if nt % 2 in desc_s:
                desc_s.pop(nt % 2).wait()
            desc_g = gather(nt)

        @pl.loop(0, CH // 2)
        def _row(r):
            rv = rows[t % 2]
            for h in range(2):
                e = r + 64 * h
                for q in range(4):
                    vi = wv[t % 2][r, pl.ds(64 * h + q * 16, 16)]
                    flo = plsc.bitcast(vi << 16, jnp.float32)
                    fhi = plsc.bitcast(vi & (-65536), jnp.float32)
                    slo = pl.ds(q * 16, 16)
                    shi = pl.ds(64 + q * 16, 16)
                    rv[e, slo] = rv[e, slo] * flo
                    rv[e, shi] = rv[e, shi] * fhi

        desc_s[t % 2] = pltpu.async_copy(rows[t % 2], agg_sh.at[dst_row(t)],
                                         ssem, add=True)
    for d in desc_s.values():
        d.wait()

    plsc.subcore_barrier()

    @pl.when(s < NSTAGE)
    def _writeout():
        pltpu.sync_copy(agg_sh.at[pl.ds(rbase, RPT), :],
                        agg_hbm.at[c, pl.ds(rbase, RPT), :])


_gms_call = pl.kernel(
    _gms_body,
    out_type=jax.ShapeDtypeStruct((NC, NN, 128), jnp.float32),
    mesh=plsc.VectorSubcoreMesh(**_SC_MESH),
    compiler_params=_SC_PARAMS,
    scratch_types=[
        pltpu.VMEM_SHARED((NN, 128), jnp.float32),
        pltpu.VMEM((IGRP, 128), jnp.int32),
        pltpu.VMEM((2 * IGRP, 128), jnp.int32),
        pltpu.VMEM((CH // 2, 128), jnp.int32),
        pltpu.VMEM((CH // 2, 128), jnp.int32),
        pltpu.VMEM((CH, 128), jnp.float32),
        pltpu.VMEM((CH, 128), jnp.float32),
        pltpu.SemaphoreType.DMA,
        pltpu.SemaphoreType.DMA,
        pltpu.SemaphoreType.DMA,
    ],
)


def _silu(x):
    return x * jax.nn.sigmoid(x)


def _rne_bf16_bits(x):
    r = lax.bitcast_convert_type(x, jnp.int32)
    return ((r + 0x7FFF + ((r >> 16) & 1)) >> 16) & 0xFFFF


def _pack_pair(wa, wb):
    return _rne_bf16_bits(wa) | (_rne_bf16_bits(wb) << 16)


_RB = 2048


def _basis(d2):
    el = jnp.sqrt(d2 + 1e-12)
    kk = (lax.broadcasted_iota(jnp.int32, (_RB, 16), 1) + 1).astype(jnp.float32)
    diff = el * INV_STEP - kk
    a = diff + 1.0
    b = 1.0 - diff
    ok = (a > 0.0) & (b > 0.0)
    arg = -1.0 / jnp.where(ok, a, 1.0) - 1.0 / jnp.where(ok, b, 1.0)
    return jnp.where(ok, BASIS_CONST * jnp.exp(arg), 0.0)


def _dot(a, b):
    return jnp.dot(a, b, preferred_element_type=jnp.float32)


def _head(basis, m0, m1):
    return _silu(_dot(_silu(_dot(basis, m0[...])), m1[...]))


def _pack_wide(w):
    bits = _rne_bf16_bits(w)
    return bits[:, 0:64] | (bits[:, 64:128] << 16)


def _store_packed(wref, p):
    for g in range(16):
        wref[pl.ds(g * 64, 64), 0:64] = p[g * 128:g * 128 + 64, :]
        wref[pl.ds(g * 64, 64), 64:128] = p[g * 128 + 64:g * 128 + 128, :]


def _radial_if_body(d2_ref, b00, b01, b02, b10, b11, b12, b20, b21, b2a, b2b,
                    w0_ref, w1_ref, w2a_ref, w2b_ref):
    bc = jnp.concatenate([_basis(d2_ref[:, 0:1]), _basis(d2_ref[:, 1:2])], axis=1)
    for l, (wref, b2) in enumerate(((w0_ref, b20), (w1_ref, b21))):
        b0 = (b00, b01)[l]
        b1 = (b10, b11)[l]
        h = _silu(_dot(_silu(_dot(bc, b0[...])), b1[...]))
        _store_packed(wref, _pack_wide(_dot(h, b2[...])))
    h2 = _silu(_dot(_silu(_dot(bc, b02[...])), b12[...]))
    _store_packed(w2a_ref, _pack_wide(_dot(h2, b2a[...])))
    _store_packed(w2b_ref, _pack_wide(_dot(h2, b2b[...])))


def _radial_ts_body(d2_ref, m0c, bd1, bd2, m02, m12, m22,
                    w0_ref, w1_ref, w2_ref):
    b = _basis(d2_ref[...])
    h = _silu(_dot(_silu(_dot(b, m0c[...])), bd1[...]))
    bits = _rne_bf16_bits(_dot(h, bd2[...]))
    _store_packed(w0_ref, bits[:, 0:64])
    _store_packed(w1_ref, bits[:, 64:128])
    h2 = _silu(_dot(_silu(_dot(b, m02[...])), m12[...]))
    _store_packed(w2_ref, _pack_wide(_dot(h2, m22[...])))


def _wspec(r, cc):
    return pl.BlockSpec((r, cc), lambda i: (0, 0))


_ESPEC = pl.BlockSpec((_RB // 2, 128), lambda i: (i, 0))

_radial_if_call = pl.pallas_call(
    _radial_if_body,
    grid=(EP // _RB,),
    compiler_params=_TC_PARAMS,
    in_specs=[pl.BlockSpec((_RB, 2), lambda i: (i, 0))]
    + [_wspec(32, 128)] * 3 + [_wspec(128, 128)] * 7,
    out_specs=[_ESPEC] * 4,
    out_shape=[jax.ShapeDtypeStruct((EP // 2, 128), jnp.int32)] * 4,
)

_radial_ts_call = pl.pallas_call(
    _radial_ts_body,
    grid=(EP // _RB,),
    compiler_params=_TC_PARAMS,
    in_specs=[pl.BlockSpec((_RB, 1), lambda i: (i, 0)),
              _wspec(16, 128), _wspec(128, 128), _wspec(128, 128),
              _wspec(16, 64), _wspec(64, 64), _wspec(64, 128)],
    out_specs=[_ESPEC] * 3,
    out_shape=[jax.ShapeDtypeStruct((EP // 2, 128), jnp.int32)] * 3,
)


@functools.lru_cache(maxsize=None)
def _make_y_pair(ci):
    def body(hi_ref, wi_ref, hf_ref, wf_ref, o_ref):
        o_ref[:, 0:64] = _dot(hi_ref[...], wi_ref[...])
        o_ref[:, 64:128] = _dot(hf_ref[...], wf_ref[...])

    return pl.pallas_call(
        body,
        grid=(1,),
        compiler_params=_TC_PARAMS,
        in_specs=[_wspec(NN, ci), _wspec(ci, 64), _wspec(NN, ci), _wspec(ci, 64)],
        out_specs=_wspec(NN, 128),
        out_shape=jax.ShapeDtypeStruct((NN, 128), jnp.float32),
    )


@functools.lru_cache(maxsize=None)
def _make_y_single(ci, co):
    def body(h_ref, w_ref, o_ref):
        if co == 64:
            o_ref[:, 0:64] = _dot(h_ref[...], w_ref[...])
            o_ref[:, 64:128] = jnp.zeros((NN, 64), jnp.float32)
        else:
            o_ref[...] = _dot(h_ref[...], w_ref[...])

    return pl.pallas_call(
        body,
        grid=(1,),
        compiler_params=_TC_PARAMS,
        in_specs=[_wspec(NN, ci), _wspec(ci, co)],
        out_specs=_wspec(NN, 128),
        out_shape=jax.ShapeDtypeStruct((NN, 128), jnp.float32),
    )


@functools.lru_cache(maxsize=None)
def _make_combine_pair(ci):
    def body(hi_ref, wi_ref, hf_ref, wf_ref, agg_ref, oi_ref, of_ref):
        a = (agg_ref[0] + agg_ref[1]) * INV_SQRT_NEIGH
        oi_ref[...] = _silu(_dot(hi_ref[...], wi_ref[...]) + a[:, 0:64])
        of_ref[...] = _silu(_dot(hf_ref[...], wf_ref[...]) + a[:, 64:128])

    return pl.pallas_call(
        body,
        grid=(1,),
        compiler_params=_TC_PARAMS,
        in_specs=[_wspec(NN, ci), _wspec(ci, 64), _wspec(NN, ci), _wspec(ci, 64),
                  pl.BlockSpec((NC, NN, 128), lambda i: (0, 0, 0))],
        out_specs=[_wspec(NN, 64)] * 2,
        out_shape=[jax.ShapeDtypeStruct((NN, 64), jnp.float32)] * 2,
    )


def _combine_pair_final_body(hi_ref, wi_ref, hf_ref, wf_ref, agga_ref, aggb_ref,
                             oi_ref, of_ref):
    a = (agga_ref[0] + agga_ref[1]) * INV_SQRT_NEIGH
    b = (aggb_ref[0] + aggb_ref[1]) * INV_SQRT_NEIGH
    ti = _dot(hi_ref[...], wi_ref[...])
    tf = _dot(hf_ref[...], wf_ref[...])
    oi_ref[:, 0:64] = ti[:, 0:64] + a[:, 0:64]
    oi_ref[:, 64:128] = ti[:, 64:128] + b[:, 0:64]
    of_ref[:, 0:64] = tf[:, 0:64] + a[:, 64:128]
    of_ref[:, 64:128] = tf[:, 64:128] + b[:, 64:128]


_combine_pair_final_call = pl.pallas_call(
    _combine_pair_final_body,
    grid=(1,),
        compiler_params=_TC_PARAMS,
    in_specs=[_wspec(NN, 64), _wspec(64, 128), _wspec(NN, 64), _wspec(64, 128),
              pl.BlockSpec((NC, NN, 128), lambda i: (0, 0, 0)),
              pl.BlockSpec((NC, NN, 128), lambda i: (0, 0, 0))],
    out_specs=[_wspec(NN, 128)] * 2,
    out_shape=[jax.ShapeDtypeStruct((NN, 128), jnp.float32)] * 2,
)


@functools.lru_cache(maxsize=None)
def _make_combine_single(ci, co, do_silu):
    def body(h_ref, w_ref, agg_ref, o_ref):
        a = (agg_ref[0] + agg_ref[1]) * INV_SQRT_NEIGH
        t = _dot(h_ref[...], w_ref[...]) + a[:, 0:co]
        o_ref[...] = _silu(t) if do_silu else t

    return pl.pallas_call(
        body,
        grid=(1,),
        compiler_params=_TC_PARAMS,
        in_specs=[_wspec(NN, ci), _wspec(ci, co),
                  pl.BlockSpec((NC, NN, 128), lambda i: (0, 0, 0))],
        out_specs=_wspec(NN, co),
        out_shape=jax.ShapeDtypeStruct((NN, co), jnp.float32),
    )


def _pad_m0(m0):
    return jnp.pad(m0, ((0, 16 - NUM_BASIS), (0, 0)))


def _bd(a, b):
    za = jnp.zeros((a.shape[0], b.shape[1]), a.dtype)
    zb = jnp.zeros((b.shape[0], a.shape[1]), b.dtype)
    return jnp.concatenate(
        [jnp.concatenate([a, za], axis=1), jnp.concatenate([zb, b], axis=1)], axis=0)


def _prm_if(pi, pf):
    out = [_bd(_pad_m0(pi['m0_%d' % l]), _pad_m0(pf['m0_%d' % l])) for l in range(3)]
    out += [_bd(pi['m1_%d' % l], pf['m1_%d' % l]) for l in range(3)]
    out += [_bd(pi['m2_0'], pf['m2_0']), _bd(pi['m2_1'], pf['m2_1']),
            _bd(pi['m2_2'][:, 0:64], pf['m2_2'][:, 0:64]),
            _bd(pi['m2_2'][:, 64:128], pf['m2_2'][:, 64:128])]
    return out


def _prm_ts(pt):
    return [jnp.concatenate([_pad_m0(pt['m0_0']), _pad_m0(pt['m0_1'])], axis=1),
            _bd(pt['m1_0'], pt['m1_1']), _bd(pt['m2_0'], pt['m2_1']),
            _pad_m0(pt['m0_2']), pt['m1_2'], pt['m2_2']]


def kernel(pos, x, pos_final_state, x_final_state, pos_interpolated_transition_state,
           p, species, batch, edge_index, params):
    pad_idx = (jnp.arange(EP - EE, dtype=jnp.int32) % NN)
    src = jnp.concatenate([edge_index[0].astype(jnp.int32), pad_idx]).reshape(EP // 128, 128)
    dst = jnp.concatenate([edge_index[1].astype(jnp.int32), pad_idx]).reshape(EP // 128, 128)
    zeros128 = jnp.zeros((NN, 128), jnp.float32)
    pi, pf, pt = params['init'], params['final'], params['ts']

    pt_pos = pos_interpolated_transition_state
    d2i, d2f, d2t = _d2_call(
        pos[:, 0], pos[:, 1], pos[:, 2],
        pos_final_state[:, 0], pos_final_state[:, 1], pos_final_state[:, 2],
        pt_pos[:, 0], pt_pos[:, 1], pt_pos[:, 2], src, dst)
    d2if = jnp.concatenate([d2i.reshape(EP, 1), d2f.reshape(EP, 1)], axis=1)
    d2t = d2t.reshape(EP, 1)
    w0, w1, w2a, w2b = _radial_if_call(d2if, *_prm_if(pi, pf))

    hi, hf = x, x_final_state
    tws = None
    for l, w in ((0, w0), (1, w1)):
        ci = hi.shape[1]
        y = _make_y_pair(ci)(hi, pi['W1_%d' % l], hf, pf['W1_%d' % l])
        agg = _gms_call(y, w, src, dst, zeros128)
        if tws is None:
            tws = _radial_ts_call(d2t, *_prm_ts(pt))
        hi, hf = _make_combine_pair(ci)(hi, pi['Wsc_%d' % l], hf, pf['Wsc_%d' % l], agg)
    tw0, tw1, tw2 = tws
    ya = _make_y_pair(64)(hi, pi['W1_2'][:, 0:64], hf, pf['W1_2'][:, 0:64])
    yb = _make_y_pair(64)(hi, pi['W1_2'][:, 64:128], hf, pf['W1_2'][:, 64:128])
    agga = _gms_call(ya, w2a, src, dst, zeros128)
    aggb = _gms_call(yb, w2b, src, dst, zeros128)
    out_i, out_f = _combine_pair_final_call(hi, pi['Wsc_2'], hf, pf['Wsc_2'], agga, aggb)

    h = p[0] * out_i + (1.0 - p[0]) * out_f
    for l, w in ((0, tw0), (1, tw1)):
        ci = h.shape[1]
        y = _make_y_single(ci, 64)(h, pt['W1_%d' % l])
        agg = _gms_call(y, w, src, dst, zeros128)
        h = _make_combine_single(ci, 64, True)(h, pt['Wsc_%d' % l], agg)
    y = _make_y_single(64, 128)(h, pt['W1_2'])
    agg = _gms_call(y, tw2, src, dst, zeros128)
    return _make_combine_single(64, 128, False)(h, pt['Wsc_2'], agg)


# speedup vs baseline: 1.0134x; 1.0134x over previous
import functools
import math

import jax
import jax.numpy as jnp
import numpy as np
from jax import lax
from jax.experimental import pallas as pl
from jax.experimental.pallas import tpu as pltpu
from jax.experimental.pallas import tpu_sc as plsc

NN = 10000
EE = 320000
NUM_BASIS = 10
MAX_RADIUS = 5.0
INV_SQRT_NEIGH = 1.0 / math.sqrt(32.0)
BASIS_CONST = 1.14136 * float(np.exp(2.0)) * float(np.sqrt(NUM_BASIS))
INV_STEP = (NUM_BASIS + 1) / MAX_RADIUS
PAD_D2 = 1.0e6

NC, NS = 2, 16
NW = NC * NS
EP = 327680
EPT = EP // NW
IR = EPT // 128
CH = 128
IGRP = 4
NGRP = IR // IGRP
NSTAGE = 10
RPT = NN // NSTAGE

_SC_MESH = dict(core_axis_name="c", subcore_axis_name="s")
_SC_PARAMS = pltpu.CompilerParams(needs_layout_passes=False)


def _d2_body(pxi, pyi, pzi, pxf, pyf, pzf, pxt, pyt, pzt, src_hbm, dst_hbm,
             d2i_hbm, d2f_hbm, d2t_hbm,
             px_v, py_v, pz_v, src_v, dst_v, d2_v):
    c = lax.axis_index("c")
    s = lax.axis_index("s")
    wid = c * NS + s
    base = wid * EPT
    pltpu.sync_copy(src_hbm.at[pl.ds(wid * IR, IR), :], src_v)
    pltpu.sync_copy(dst_hbm.at[pl.ds(wid * IR, IR), :], dst_v)
    for (px, py, pz, d2_hbm) in ((pxi, pyi, pzi, d2i_hbm),
                                 (pxf, pyf, pzf, d2f_hbm),
                                 (pxt, pyt, pzt, d2t_hbm)):
        pltpu.sync_copy(px, px_v)
        pltpu.sync_copy(py, py_v)
        pltpu.sync_copy(pz, pz_v)

        @pl.loop(0, IR)
        def _row(r):
            @pl.loop(0, 8)
            def _grp(q):
                si = src_v[r, pl.ds(q * 16, 16)]
                di = dst_v[r, pl.ds(q * 16, 16)]
                dx = plsc.load_gather(px_v, [si]) - plsc.load_gather(px_v, [di])
                dy = plsc.load_gather(py_v, [si]) - plsc.load_gather(py_v, [di])
                dz = plsc.load_gather(pz_v, [si]) - plsc.load_gather(pz_v, [di])
                d2 = dx * dx + dy * dy + dz * dz
                off = r * 128 + q * 16
                gidx = lax.iota(jnp.int32, 16) + (base + off)
                d2_v[pl.ds(off, 16)] = jnp.where(gidx >= EE, PAD_D2, d2)

        pltpu.sync_copy(d2_v, d2_hbm.at[pl.ds(base, EPT)])


_d2_call = pl.kernel(
    _d2_body,
    out_type=[jax.ShapeDtypeStruct((EP,), jnp.float32)] * 3,
    mesh=plsc.VectorSubcoreMesh(**_SC_MESH),
    compiler_params=_SC_PARAMS,
    scratch_types=[
        pltpu.VMEM((NN,), jnp.float32),
        pltpu.VMEM((NN,), jnp.float32),
        pltpu.VMEM((NN,), jnp.float32),
        pltpu.VMEM((IR, 128), jnp.int32),
        pltpu.VMEM((IR, 128), jnp.int32),
        pltpu.VMEM((EPT,), jnp.float32),
    ],
)


def _gms_body(y_hbm, w_hbm, src_hbm, dst_hbm, zeros_hbm, agg_hbm,
              agg_sh, src_v, dst_v, w_a, w_b, rows_a, rows_b, gsem, ssem, wsem):
    c = lax.axis_index("c")
    s = lax.axis_index("s")
    wid = c * NS + s
    base = wid * EPT
    rbase = s * RPT
    @pl.when(s < NSTAGE)
    def _zero():
        pltpu.sync_copy(zeros_hbm.at[pl.ds(rbase, RPT), :],
                        agg_sh.at[pl.ds(rbase, RPT), :])
    plsc.subcore_barrier()

    rows = (rows_a, rows_b)
    wv = (w_a, w_b)
    nchunk = IR

    def stage_idx(g):
        pltpu.sync_copy(src_hbm.at[pl.ds(wid * IR + g * IGRP, IGRP), :], src_v)
        pltpu.sync_copy(dst_hbm.at[pl.ds(wid * IR + g * IGRP, IGRP), :],
                        dst_v.at[pl.ds((g % 2) * IGRP, IGRP), :])

    def gather(t):
        return pltpu.async_copy(y_hbm.at[src_v.at[t % IGRP]], rows[t % 2], gsem)

    def dst_row(t):
        return dst_v.at[((t // IGRP) % 2) * IGRP + (t % IGRP)]

    def wstream(t):
        off = pl.multiple_of(base // 2 + t * (CH // 2), 8)
        return pltpu.async_copy(
            w_hbm.at[pl.ds(off, CH // 2), :], wv[t % 2], wsem)

    stage_idx(0)
    desc_w = wstream(0)
    desc_g = gather(0)
    desc_s = {}
    for t in range(nchunk):
        desc_w.wait()
        desc_g.wait()
        nt = t + 1
        if nt < nchunk:
            if nt % IGRP == 0:
                stage_idx(nt // IGRP)
            desc_w = wstream(nt)
            if nt % 2 in desc_s:
                desc_s.pop(nt % 2).wait()
            desc_g = gather(nt)

        @pl.loop(0, CH // 2)
        def _row(r):
            rv = rows[t % 2]
            for h in range(2):
                e = r + 64 * h
                for q in range(4):
                    vi = wv[t % 2][r, pl.ds(64 * h + q * 16, 16)]
                    flo = plsc.bitcast(vi << 16, jnp.float32)
                    fhi = plsc.bitcast(vi & (-65536), jnp.float32)
                    slo = pl.ds(q * 16, 16)
                    shi = pl.ds(64 + q * 16, 16)
                    rv[e, slo] = rv[e, slo] * flo
                    rv[e, shi] = rv[e, shi] * fhi

        desc_s[t % 2] = pltpu.async_copy(rows[t % 2], agg_sh.at[dst_row(t)],
                                         ssem, add=True)
    for d in desc_s.values():
        d.wait()

    plsc.subcore_barrier()

    @pl.when(s < NSTAGE)
    def _writeout():
        pltpu.sync_copy(agg_sh.at[pl.ds(rbase, RPT), :],
                        agg_hbm.at[c, pl.ds(rbase, RPT), :])


_gms_call = pl.kernel(
    _gms_body,
    out_type=jax.ShapeDtypeStruct((NC, NN, 128), jnp.float32),
    mesh=plsc.VectorSubcoreMesh(**_SC_MESH),
    compiler_params=_SC_PARAMS,
    scratch_types=[
        pltpu.VMEM_SHARED((NN, 128), jnp.float32),
        pltpu.VMEM((IGRP, 128), jnp.int32),
        pltpu.VMEM((2 * IGRP, 128), jnp.int32),
        pltpu.VMEM((CH // 2, 128), jnp.int32),
        pltpu.VMEM((CH // 2, 128), jnp.int32),
        pltpu.VMEM((CH, 128), jnp.float32),
        pltpu.VMEM((CH, 128), jnp.float32),
        pltpu.SemaphoreType.DMA,
        pltpu.SemaphoreType.DMA,
        pltpu.SemaphoreType.DMA,
    ],
)


def _silu(x):
    return x * jax.nn.sigmoid(x)


def _rne_bf16_bits(x):
    r = lax.bitcast_convert_type(x, jnp.int32)
    return ((r + 0x7FFF + ((r >> 16) & 1)) >> 16) & 0xFFFF


def _pack_pair(wa, wb):
    return _rne_bf16_bits(wa) | (_rne_bf16_bits(wb) << 16)


_RB = 2048


def _basis(d2):
    el = jnp.sqrt(d2 + 1e-12)
    kk = (lax.broadcasted_iota(jnp.int32, (_RB, 16), 1) + 1).astype(jnp.float32)
    diff = el * INV_STEP - kk
    a = diff + 1.0
    b = 1.0 - diff
    ok = (a > 0.0) & (b > 0.0)
    arg = -2.0 / jnp.where(ok, a * b, 1.0)
    return jnp.where(ok, BASIS_CONST * jnp.exp(arg), 0.0)


def _dot(a, b):
    return jnp.dot(a, b, preferred_element_type=jnp.float32)


def _head(basis, m0, m1):
    return _silu(_dot(_silu(_dot(basis, m0[...])), m1[...]))


def _pack_wide(w):
    bits = _rne_bf16_bits(w)
    return bits[:, 0:64] | (bits[:, 64:128] << 16)


def _store_packed(wref, p):
    for g in range(16):
        wref[pl.ds(g * 64, 64), 0:64] = p[g * 128:g * 128 + 64, :]
        wref[pl.ds(g * 64, 64), 64:128] = p[g * 128 + 64:g * 128 + 128, :]


def _radial_if_body(d2_ref, b00, b01, b02, b10, b11, b12, b20, b21, b2a, b2b,
                    w0_ref, w1_ref, w2a_ref, w2b_ref):
    bc = jnp.concatenate([_basis(d2_ref[:, 0:1]), _basis(d2_ref[:, 1:2])], axis=1)
    for l, (wref, b2) in enumerate(((w0_ref, b20), (w1_ref, b21))):
        b0 = (b00, b01)[l]
        b1 = (b10, b11)[l]
        h = _silu(_dot(_silu(_dot(bc, b0[...])), b1[...]))
        _store_packed(wref, _pack_wide(_dot(h, b2[...])))
    h2 = _silu(_dot(_silu(_dot(bc, b02[...])), b12[...]))
    _store_packed(w2a_ref, _pack_wide(_dot(h2, b2a[...])))
    _store_packed(w2b_ref, _pack_wide(_dot(h2, b2b[...])))


def _radial_ts_body(d2_ref, m0c, bd1, bd2, m02, m12, m22,
                    w0_ref, w1_ref, w2_ref):
    b = _basis(d2_ref[...])
    h = _silu(_dot(_silu(_dot(b, m0c[...])), bd1[...]))
    bits = _rne_bf16_bits(_dot(h, bd2[...]))
    _store_packed(w0_ref, bits[:, 0:64])
    _store_packed(w1_ref, bits[:, 64:128])
    h2 = _silu(_dot(_silu(_dot(b, m02[...])), m12[...]))
    _store_packed(w2_ref, _pack_wide(_dot(h2, m22[...])))


def _wspec(r, cc):
    return pl.BlockSpec((r, cc), lambda i: (0, 0))


_ESPEC = pl.BlockSpec((_RB // 2, 128), lambda i: (i, 0))

_radial_if_call = pl.pallas_call(
    _radial_if_body,
    grid=(EP // _RB,),
    in_specs=[pl.BlockSpec((_RB, 2), lambda i: (i, 0))]
    + [_wspec(32, 128)] * 3 + [_wspec(128, 128)] * 7,
    out_specs=[_ESPEC] * 4,
    out_shape=[jax.ShapeDtypeStruct((EP // 2, 128), jnp.int32)] * 4,
)

_radial_ts_call = pl.pallas_call(
    _radial_ts_body,
    grid=(EP // _RB,),
    in_specs=[pl.BlockSpec((_RB, 1), lambda i: (i, 0)),
              _wspec(16, 128), _wspec(128, 128), _wspec(128, 128),
              _wspec(16, 64), _wspec(64, 64), _wspec(64, 128)],
    out_specs=[_ESPEC] * 3,
    out_shape=[jax.ShapeDtypeStruct((EP // 2, 128), jnp.int32)] * 3,
)


@functools.lru_cache(maxsize=None)
def _make_y_pair(ci):
    def body(hi_ref, wi_ref, hf_ref, wf_ref, o_ref):
        o_ref[:, 0:64] = _dot(hi_ref[...], wi_ref[...])
        o_ref[:, 64:128] = _dot(hf_ref[...], wf_ref[...])

    return pl.pallas_call(
        body,
        grid=(1,),
        in_specs=[_wspec(NN, ci), _wspec(ci, 64), _wspec(NN, ci), _wspec(ci, 64)],
        out_specs=_wspec(NN, 128),
        out_shape=jax.ShapeDtypeStruct((NN, 128), jnp.float32),
    )


@functools.lru_cache(maxsize=None)
def _make_y_single(ci, co):
    def body(h_ref, w_ref, o_ref):
        if co == 64:
            o_ref[:, 0:64] = _dot(h_ref[...], w_ref[...])
            o_ref[:, 64:128] = jnp.zeros((NN, 64), jnp.float32)
        else:
            o_ref[...] = _dot(h_ref[...], w_ref[...])

    return pl.pallas_call(
        body,
        grid=(1,),
        in_specs=[_wspec(NN, ci), _wspec(ci, co)],
        out_specs=_wspec(NN, 128),
        out_shape=jax.ShapeDtypeStruct((NN, 128), jnp.float32),
    )


@functools.lru_cache(maxsize=None)
def _make_combine_pair(ci):
    def body(hi_ref, wi_ref, hf_ref, wf_ref, agg_ref, oi_ref, of_ref):
        a = (agg_ref[0] + agg_ref[1]) * INV_SQRT_NEIGH
        oi_ref[...] = _silu(_dot(hi_ref[...], wi_ref[...]) + a[:, 0:64])
        of_ref[...] = _silu(_dot(hf_ref[...], wf_ref[...]) + a[:, 64:128])

    return pl.pallas_call(
        body,
        grid=(1,),
        in_specs=[_wspec(NN, ci), _wspec(ci, 64), _wspec(NN, ci), _wspec(ci, 64),
                  pl.BlockSpec((NC, NN, 128), lambda i: (0, 0, 0))],
        out_specs=[_wspec(NN, 64)] * 2,
        out_shape=[jax.ShapeDtypeStruct((NN, 64), jnp.float32)] * 2,
    )


def _combine_pair_final_body(hi_ref, wi_ref, hf_ref, wf_ref, agga_ref, aggb_ref,
                             oi_ref, of_ref):
    a = (agga_ref[0] + agga_ref[1]) * INV_SQRT_NEIGH
    b = (aggb_ref[0] + aggb_ref[1]) * INV_SQRT_NEIGH
    ti = _dot(hi_ref[...], wi_ref[...])
    tf = _dot(hf_ref[...], wf_ref[...])
    oi_ref[:, 0:64] = ti[:, 0:64] + a[:, 0:64]
    oi_ref[:, 64:128] = ti[:, 64:128] + b[:, 0:64]
    of_ref[:, 0:64] = tf[:, 0:64] + a[:, 64:128]
    of_ref[:, 64:128] = tf[:, 64:128] + b[:, 64:128]


_combine_pair_final_call = pl.pallas_call(
    _combine_pair_final_body,
    grid=(1,),
    in_specs=[_wspec(NN, 64), _wspec(64, 128), _wspec(NN, 64), _wspec(64, 128),
              pl.BlockSpec((NC, NN, 128), lambda i: (0, 0, 0)),
              pl.BlockSpec((NC, NN, 128), lambda i: (0, 0, 0))],
    out_specs=[_wspec(NN, 128)] * 2,
    out_shape=[jax.ShapeDtypeStruct((NN, 128), jnp.float32)] * 2,
)


@functools.lru_cache(maxsize=None)
def _make_combine_single(ci, co, do_silu):
    def body(h_ref, w_ref, agg_ref, o_ref):
        a = (agg_ref[0] + agg_ref[1]) * INV_SQRT_NEIGH
        t = _dot(h_ref[...], w_ref[...]) + a[:, 0:co]
        o_ref[...] = _silu(t) if do_silu else t

    return pl.pallas_call(
        body,
        grid=(1,),
        in_specs=[_wspec(NN, ci), _wspec(ci, co),
                  pl.BlockSpec((NC, NN, 128), lambda i: (0, 0, 0))],
        out_specs=_wspec(NN, co),
        out_shape=jax.ShapeDtypeStruct((NN, co), jnp.float32),
    )


def _pad_m0(m0):
    return jnp.pad(m0, ((0, 16 - NUM_BASIS), (0, 0)))


def _bd(a, b):
    za = jnp.zeros((a.shape[0], b.shape[1]), a.dtype)
    zb = jnp.zeros((b.shape[0], a.shape[1]), b.dtype)
    return jnp.concatenate(
        [jnp.concatenate([a, za], axis=1), jnp.concatenate([zb, b], axis=1)], axis=0)


def _prm_if(pi, pf):
    out = [_bd(_pad_m0(pi['m0_%d' % l]), _pad_m0(pf['m0_%d' % l])) for l in range(3)]
    out += [_bd(pi['m1_%d' % l], pf['m1_%d' % l]) for l in range(3)]
    out += [_bd(pi['m2_0'], pf['m2_0']), _bd(pi['m2_1'], pf['m2_1']),
            _bd(pi['m2_2'][:, 0:64], pf['m2_2'][:, 0:64]),
            _bd(pi['m2_2'][:, 64:128], pf['m2_2'][:, 64:128])]
    return out


def _prm_ts(pt):
    return [jnp.concatenate([_pad_m0(pt['m0_0']), _pad_m0(pt['m0_1'])], axis=1),
            _bd(pt['m1_0'], pt['m1_1']), _bd(pt['m2_0'], pt['m2_1']),
            _pad_m0(pt['m0_2']), pt['m1_2'], pt['m2_2']]


def kernel(pos, x, pos_final_state, x_final_state, pos_interpolated_transition_state,
           p, species, batch, edge_index, params):
    pad_idx = (jnp.arange(EP - EE, dtype=jnp.int32) % NN)
    src = jnp.concatenate([edge_index[0].astype(jnp.int32), pad_idx]).reshape(EP // 128, 128)
    dst = jnp.concatenate([edge_index[1].astype(jnp.int32), pad_idx]).reshape(EP // 128, 128)
    zeros128 = jnp.zeros((NN, 128), jnp.float32)
    pi, pf, pt = params['init'], params['final'], params['ts']

    pt_pos = pos_interpolated_transition_state
    d2i, d2f, d2t = _d2_call(
        pos[:, 0], pos[:, 1], pos[:, 2],
        pos_final_state[:, 0], pos_final_state[:, 1], pos_final_state[:, 2],
        pt_pos[:, 0], pt_pos[:, 1], pt_pos[:, 2], src, dst)
    d2if = jnp.concatenate([d2i.reshape(EP, 1), d2f.reshape(EP, 1)], axis=1)
    d2t = d2t.reshape(EP, 1)
    w0, w1, w2a, w2b = _radial_if_call(d2if, *_prm_if(pi, pf))

    hi, hf = x, x_final_state
    tws = None
    for l, w in ((0, w0), (1, w1)):
        ci = hi.shape[1]
        y = _make_y_pair(ci)(hi, pi['W1_%d' % l], hf, pf['W1_%d' % l])
        agg = _gms_call(y, w, src, dst, zeros128)
        if tws is None:
            tws = _radial_ts_call(d2t, *_prm_ts(pt))
        hi, hf = _make_combine_pair(ci)(hi, pi['Wsc_%d' % l], hf, pf['Wsc_%d' % l], agg)
    tw0, tw1, tw2 = tws
    ya = _make_y_pair(64)(hi, pi['W1_2'][:, 0:64], hf, pf['W1_2'][:, 0:64])
    yb = _make_y_pair(64)(hi, pi['W1_2'][:, 64:128], hf, pf['W1_2'][:, 64:128])
    agga = _gms_call(ya, w2a, src, dst, zeros128)
    aggb = _gms_call(yb, w2b, src, dst, zeros128)
    out_i, out_f = _combine_pair_final_call(hi, pi['Wsc_2'], hf, pf['Wsc_2'], agga, aggb)

    h = p[0] * out_i + (1.0 - p[0]) * out_f
    for l, w in ((0, tw0), (1, tw1)):
        ci = h.shape[1]
        y = _make_y_single(ci, 64)(h, pt['W1_%d' % l])
        agg = _gms_call(y, w, src, dst, zeros128)
        h = _make_combine_single(ci, 64, True)(h, pt['Wsc_%d' % l], agg)
    y = _make_y_single(64, 128)(h, pt['W1_2'])
    agg = _gms_call(y, tw2, src, dst, zeros128)
    return _make_combine_single(64, 128, False)(h, pt['Wsc_2'], agg)
